# baseline (speedup 1.0000x reference)
import jax
import jax.numpy as jnp
from jax import lax
from jax.experimental import pallas as pl
from jax.experimental.pallas import tpu as pltpu
from jax.experimental.pallas import tpu_sc as plsc

_N = 50000
_D = 256
_G = 64
_L = 16
_NV = _D // _L
_B = 1000
_K = 3
_NBLK = _N // _B
_NBT = _NBLK // _K
_C = 40
_PB = _B // _C
_PER = _K * _PB
_OWN = (_K - 1) * _PB
_NEG = float("-inf")


def _lane_extract(vec, lane):
    lid = lax.broadcasted_iota(jnp.int32, (_L,), 0)
    return jnp.max(jnp.where(lid == lane, vec, 0))


def _sc_body(x_hbm, batch_hbm, out_hbm, batch_v, buf0, buf1, accv, sem0, sem1):
    wid = lax.axis_index("s") * 2 + lax.axis_index("c")

    pltpu.sync_copy(batch_hbm, batch_v)
    lid = lax.broadcasted_iota(jnp.int32, (_L,), 0)
    tgt = 2 * wid + lid
    lo = jnp.zeros((_L,), jnp.int32)
    hi = jnp.full((_L,), _N, jnp.int32)

    def bs_step(_, carry):
        lo, hi = carry
        active = lo < hi
        mid = lax.shift_right_logical(lo + hi, 1)
        v = plsc.load_gather(batch_v, [jnp.minimum(mid, _N - 1)])
        less = v < tgt
        lo = jnp.where(active & less, mid + 1, lo)
        hi = jnp.where(active & (~less), mid, hi)
        return lo, hi

    lo, _ = lax.fori_loop(0, 16, bs_step, (lo, hi))
    s0 = _lane_extract(lo, 0)
    sm = _lane_extract(lo, 1)
    s1 = _lane_extract(lo, 2)

    def f(c):
        return _OWN * (c // _PER) + jnp.minimum(c % _PER, _OWN)

    j0 = f(s0 // _C)
    nchunks = f((s1 + _C - 1) // _C) - j0

    def off(q):
        j = j0 + q
        c = _PER * (j // _OWN) + j % _OWN
        return pl.multiple_of(c * _C, 8)

    def start(k, buf, sem):
        pltpu.make_async_copy(x_hbm.at[pl.ds(off(k), _C)], buf, sem).start()

    @pl.when(nchunks > 0)
    def _p0():
        start(0, buf0, sem0)

    @pl.when(nchunks > 1)
    def _p1():
        start(1, buf1, sem1)

    neg = jnp.full((_L,), _NEG, jnp.float32)
    acc = [neg] * (2 * _NV)

    def make_row_body(buf, base):
        def row_body(r, a):
            a = list(a)
            for j in range(_NV):
                a[base + j] = jnp.maximum(a[base + j], buf[r, pl.ds(j * _L, _L)])
            return tuple(a)
        return row_body

    def chunk(k, buf, sem, acc):
        use = k < nchunks

        @pl.when(use)
        def _w():
            pltpu.make_async_copy(
                x_hbm.at[pl.ds(off(k), _C)], buf, sem).wait()

        o = off(k)
        i0 = jnp.where(use, jnp.clip(s0 - o, 0, _C), 0)
        i1 = jnp.where(use, jnp.clip(sm - o, 0, _C), 0)
        i2 = jnp.where(use, jnp.clip(s1 - o, 0, _C), 0)
        acc = lax.fori_loop(i0, i1, make_row_body(buf, 0), acc)
        acc = lax.fori_loop(i1, i2, make_row_body(buf, _NV), acc)

        @pl.when(k + 2 < nchunks)
        def _n():
            start(k + 2, buf, sem)

        return acc

    def pair(p, acc):
        acc = chunk(2 * p, buf0, sem0, acc)
        acc = chunk(2 * p + 1, buf1, sem1, acc)
        return acc

    acc = lax.fori_loop(0, (nchunks + 1) // 2, pair, tuple(acc))

    for j in range(_NV):
        accv[pl.ds(j * _L, _L)] = acc[j]
        accv[pl.ds(_D + j * _L, _L)] = acc[_NV + j]
    pltpu.sync_copy(accv, out_hbm.at[pl.ds(wid * 2 * _D, 2 * _D)])


def _tc_body(batch_ref, x_ref, out_ref):
    i = pl.program_id(0)

    @pl.when(i == 0)
    def _init():
        out_ref[...] = jnp.full((_G, _D), _NEG, dtype=jnp.float32)

    lo = batch_ref[0, 0, 0]
    hi = batch_ref[0, 0, _B - 1]
    x = x_ref[...]
    riota = lax.broadcasted_iota(jnp.int32, (_B, 1), 0)

    def first_ge(g):
        def step(_, lohi):
            l, h = lohi
            m = jnp.minimum((l + h) // 2, _B - 1)
            ge = batch_ref[0, 0, m] >= g
            return jnp.where(ge, l, m + 1), jnp.where(ge, m, h)
        l, _ = lax.fori_loop(0, 10, step, (0, _B))
        return l

    def seg_update(g, fs):
        fe = first_ge(g + 1)
        m = jnp.where((riota >= fs) & (riota < fe), x, _NEG).max(
            axis=0, keepdims=True)
        cur = out_ref[pl.ds(g, 1), :]
        out_ref[pl.ds(g, 1), :] = jnp.maximum(cur, m)
        return fe

    lax.fori_loop(lo, hi + 1, seg_update, first_ge(lo))


@jax.jit
def kernel(x, batch):
    mesh = plsc.VectorSubcoreMesh(core_axis_name="c", subcore_axis_name="s")
    sc = pl.kernel(
        _sc_body,
        out_type=jax.ShapeDtypeStruct((_G * _D,), jnp.float32),
        mesh=mesh,
        compiler_params=pltpu.CompilerParams(needs_layout_passes=False),
        scratch_types=[
            pltpu.VMEM((_N,), jnp.int32),
            pltpu.VMEM((_C, _D), jnp.float32),
            pltpu.VMEM((_C, _D), jnp.float32),
            pltpu.VMEM((2 * _D,), jnp.float32),
            pltpu.SemaphoreType.DMA,
            pltpu.SemaphoreType.DMA,
        ],
    )
    sc_out = sc(x, batch).reshape(_G, _D)

    batch3 = batch.reshape(_NBLK, 1, _B)
    tc_out = pl.pallas_call(
        _tc_body,
        grid=(_NBT,),
        in_specs=[
            pl.BlockSpec((1, 1, _B), lambda i: (_K * i + _K - 1, 0, 0),
                         memory_space=pltpu.SMEM),
            pl.BlockSpec((_B, _D), lambda i: (_K * i + _K - 1, 0)),
        ],
        out_specs=pl.BlockSpec((_G, _D), lambda i: (0, 0)),
        out_shape=jax.ShapeDtypeStruct((_G, _D), jnp.float32),
    )(batch3, x)

    return jnp.maximum(sc_out, tc_out)


# speedup vs baseline: 1.0962x; 1.0962x over previous
import jax
import jax.numpy as jnp
from jax import lax
from jax.experimental import pallas as pl
from jax.experimental.pallas import tpu as pltpu
from jax.experimental.pallas import tpu_sc as plsc

_N = 50000
_D = 256
_G = 64
_L = 16
_NV = _D // _L
_B = 2000
_K = 3
_NBLK = _N // _B
_NBT = _NBLK // _K
_C = 80
_PB = _B // _C
_PER = _K * _PB
_OWN = (_K - 1) * _PB
_NEG = float("-inf")


def _lane_extract(vec, lane):
    lid = lax.broadcasted_iota(jnp.int32, (_L,), 0)
    return jnp.max(jnp.where(lid == lane, vec, 0))


def _sc_body(x_hbm, batch_hbm, out_hbm, batch_v, buf0, buf1, accv, sem0, sem1):
    wid = lax.axis_index("s") * 2 + lax.axis_index("c")

    pltpu.sync_copy(batch_hbm, batch_v)
    lid = lax.broadcasted_iota(jnp.int32, (_L,), 0)
    tgt = 2 * wid + lid
    lo = jnp.zeros((_L,), jnp.int32)
    hi = jnp.full((_L,), _N, jnp.int32)

    def bs_step(_, carry):
        lo, hi = carry
        active = lo < hi
        mid = lax.shift_right_logical(lo + hi, 1)
        v = plsc.load_gather(batch_v, [jnp.minimum(mid, _N - 1)])
        less = v < tgt
        lo = jnp.where(active & less, mid + 1, lo)
        hi = jnp.where(active & (~less), mid, hi)
        return lo, hi

    lo, _ = lax.fori_loop(0, 16, bs_step, (lo, hi))
    s0 = _lane_extract(lo, 0)
    sm = _lane_extract(lo, 1)
    s1 = _lane_extract(lo, 2)

    def f(c):
        return _OWN * (c // _PER) + jnp.minimum(c % _PER, _OWN)

    j0 = f(s0 // _C)
    nchunks = f((s1 + _C - 1) // _C) - j0

    def off(q):
        j = j0 + q
        c = _PER * (j // _OWN) + j % _OWN
        return pl.multiple_of(c * _C, 8)

    def start(k, buf, sem):
        pltpu.make_async_copy(x_hbm.at[pl.ds(off(k), _C)], buf, sem).start()

    @pl.when(nchunks > 0)
    def _p0():
        start(0, buf0, sem0)

    @pl.when(nchunks > 1)
    def _p1():
        start(1, buf1, sem1)

    neg = jnp.full((_L,), _NEG, jnp.float32)
    acc = [neg] * (2 * _NV)

    def make_row_body(buf, base):
        def row_body(r, a):
            a = list(a)
            for j in range(_NV):
                a[base + j] = jnp.maximum(a[base + j], buf[r, pl.ds(j * _L, _L)])
            return tuple(a)
        return row_body

    def chunk(k, buf, sem, acc):
        use = k < nchunks

        @pl.when(use)
        def _w():
            pltpu.make_async_copy(
                x_hbm.at[pl.ds(off(k), _C)], buf, sem).wait()

        o = off(k)
        i0 = jnp.where(use, jnp.clip(s0 - o, 0, _C), 0)
        i1 = jnp.where(use, jnp.clip(sm - o, 0, _C), 0)
        i2 = jnp.where(use, jnp.clip(s1 - o, 0, _C), 0)
        acc = lax.fori_loop(i0, i1, make_row_body(buf, 0), acc)
        acc = lax.fori_loop(i1, i2, make_row_body(buf, _NV), acc)

        @pl.when(k + 2 < nchunks)
        def _n():
            start(k + 2, buf, sem)

        return acc

    def pair(p, acc):
        acc = chunk(2 * p, buf0, sem0, acc)
        acc = chunk(2 * p + 1, buf1, sem1, acc)
        return acc

    acc = lax.fori_loop(0, (nchunks + 1) // 2, pair, tuple(acc))

    for j in range(_NV):
        accv[pl.ds(j * _L, _L)] = acc[j]
        accv[pl.ds(_D + j * _L, _L)] = acc[_NV + j]
    pltpu.sync_copy(accv, out_hbm.at[pl.ds(wid * 2 * _D, 2 * _D)])


def _tc_body(batch_ref, x_ref, out_ref):
    i = pl.program_id(0)

    @pl.when(i == 0)
    def _init():
        out_ref[...] = jnp.full((_G, _D), _NEG, dtype=jnp.float32)

    lo = batch_ref[0, 0, 0]
    hi = batch_ref[0, 0, _B - 1]
    x = x_ref[...]
    riota = lax.broadcasted_iota(jnp.int32, (_B, 1), 0)

    def first_ge(g):
        def step(_, lohi):
            l, h = lohi
            m = jnp.minimum((l + h) // 2, _B - 1)
            ge = batch_ref[0, 0, m] >= g
            return jnp.where(ge, l, m + 1), jnp.where(ge, m, h)
        l, _ = lax.fori_loop(0, 11, step, (0, _B))
        return l

    def seg_update(g, fs):
        fe = first_ge(g + 1)
        m = jnp.where((riota >= fs) & (riota < fe), x, _NEG).max(
            axis=0, keepdims=True)
        cur = out_ref[pl.ds(g, 1), :]
        out_ref[pl.ds(g, 1), :] = jnp.maximum(cur, m)
        return fe

    lax.fori_loop(lo, hi + 1, seg_update, first_ge(lo))


@jax.jit
def kernel(x, batch):
    mesh = plsc.VectorSubcoreMesh(core_axis_name="c", subcore_axis_name="s")
    sc = pl.kernel(
        _sc_body,
        out_type=jax.ShapeDtypeStruct((_G * _D,), jnp.float32),
        mesh=mesh,
        compiler_params=pltpu.CompilerParams(needs_layout_passes=False),
        scratch_types=[
            pltpu.VMEM((_N,), jnp.int32),
            pltpu.VMEM((_C, _D), jnp.float32),
            pltpu.VMEM((_C, _D), jnp.float32),
            pltpu.VMEM((2 * _D,), jnp.float32),
            pltpu.SemaphoreType.DMA,
            pltpu.SemaphoreType.DMA,
        ],
    )
    sc_out = sc(x, batch).reshape(_G, _D)

    batch3 = batch.reshape(_NBLK, 1, _B)
    tc_out = pl.pallas_call(
        _tc_body,
        grid=(_NBT,),
        in_specs=[
            pl.BlockSpec((1, 1, _B), lambda i: (_K * i + _K - 1, 0, 0),
                         memory_space=pltpu.SMEM),
            pl.BlockSpec((_B, _D), lambda i: (_K * i + _K - 1, 0)),
        ],
        out_specs=pl.BlockSpec((_G, _D), lambda i: (0, 0)),
        out_shape=jax.ShapeDtypeStruct((_G, _D), jnp.float32),
    )(batch3, x)

    return jnp.maximum(sc_out, tc_out)


# speedup vs baseline: 1.1754x; 1.0723x over previous
import jax
import jax.numpy as jnp
from jax import lax
from jax.experimental import pallas as pl
from jax.experimental.pallas import tpu as pltpu
from jax.experimental.pallas import tpu_sc as plsc

_N = 50000
_D = 256
_G = 64
_L = 16
_NV = _D // _L
_C = 144
_NEG = float("-inf")


def _lane_extract(vec, lane):
    lid = lax.broadcasted_iota(jnp.int32, (_L,), 0)
    return jnp.max(jnp.where(lid == lane, vec, 0))


def _sc_body(x_hbm, batch_hbm, out_hbm, batch_v, buf0, buf1, accv, sem0, sem1):
    wid = lax.axis_index("s") * 2 + lax.axis_index("c")

    pltpu.sync_copy(batch_hbm, batch_v)
    lid = lax.broadcasted_iota(jnp.int32, (_L,), 0)
    tgt = 2 * wid + lid
    lo = jnp.zeros((_L,), jnp.int32)
    hi = jnp.full((_L,), _N, jnp.int32)

    def bs_step(_, carry):
        lo, hi = carry
        active = lo < hi
        mid = lax.shift_right_logical(lo + hi, 1)
        v = plsc.load_gather(batch_v, [jnp.minimum(mid, _N - 1)])
        less = v < tgt
        lo = jnp.where(active & less, mid + 1, lo)
        hi = jnp.where(active & (~less), mid, hi)
        return lo, hi

    lo, _ = lax.fori_loop(0, 16, bs_step, (lo, hi))
    s0 = _lane_extract(lo, 0)
    sm = _lane_extract(lo, 1)
    s1 = _lane_extract(lo, 2)

    a0 = jnp.bitwise_and(s0, -8)
    top = jnp.bitwise_and(s1 + 7, -8)
    nchunks = (top - a0 + _C - 1) // _C

    def off(k):
        o = jnp.maximum(0, jnp.minimum(a0 + k * _C, top - _C))
        return pl.multiple_of(o, 8)

    def start(k, buf, sem):
        pltpu.make_async_copy(x_hbm.at[pl.ds(off(k), _C)], buf, sem).start()

    @pl.when(nchunks > 0)
    def _p0():
        start(0, buf0, sem0)

    @pl.when(nchunks > 1)
    def _p1():
        start(1, buf1, sem1)

    neg = jnp.full((_L,), _NEG, jnp.float32)
    acc = [neg] * (2 * _NV)

    def seg_rows(buf, base, lo, hi, acc):
        def pair_body(t, a):
            a = list(a)
            for u in range(2):
                for j in range(_NV):
                    a[base + j] = jnp.maximum(
                        a[base + j], buf[lo + 2 * t + u, pl.ds(j * _L, _L)])
            return tuple(a)

        def tail_body(r, a):
            a = list(a)
            for j in range(_NV):
                a[base + j] = jnp.maximum(a[base + j], buf[r, pl.ds(j * _L, _L)])
            return tuple(a)

        half = (hi - lo) // 2
        acc = lax.fori_loop(0, half, pair_body, acc)
        return lax.fori_loop(lo + 2 * half, hi, tail_body, acc)

    def chunk(k, buf, sem, acc):
        @pl.when(k < nchunks)
        def _w():
            pltpu.make_async_copy(
                x_hbm.at[pl.ds(off(k), _C)], buf, sem).wait()

        valid = k < nchunks
        o = off(k)
        i0 = jnp.where(valid, jnp.clip(s0 - o, 0, _C), 0)
        i1 = jnp.where(valid, jnp.clip(sm - o, 0, _C), 0)
        i2 = jnp.where(valid, jnp.clip(s1 - o, 0, _C), 0)
        acc = seg_rows(buf, 0, i0, i1, acc)
        acc = seg_rows(buf, _NV, i1, i2, acc)

        @pl.when(k + 2 < nchunks)
        def _n():
            start(k + 2, buf, sem)

        return acc

    def pair(p, acc):
        acc = chunk(2 * p, buf0, sem0, acc)
        acc = chunk(2 * p + 1, buf1, sem1, acc)
        return acc

    acc = lax.fori_loop(0, (nchunks + 1) // 2, pair, tuple(acc))

    for j in range(_NV):
        accv[pl.ds(j * _L, _L)] = acc[j]
        accv[pl.ds(_D + j * _L, _L)] = acc[_NV + j]
    pltpu.sync_copy(accv, out_hbm.at[pl.ds(wid * 2 * _D, 2 * _D)])


@jax.jit
def kernel(x, batch):
    mesh = plsc.VectorSubcoreMesh(core_axis_name="c", subcore_axis_name="s")
    f = pl.kernel(
        _sc_body,
        out_type=jax.ShapeDtypeStruct((_G * _D,), jnp.float32),
        mesh=mesh,
        compiler_params=pltpu.CompilerParams(needs_layout_passes=False),
        scratch_types=[
            pltpu.VMEM((_N,), jnp.int32),
            pltpu.VMEM((_C, _D), jnp.float32),
            pltpu.VMEM((_C, _D), jnp.float32),
            pltpu.VMEM((2 * _D,), jnp.float32),
            pltpu.SemaphoreType.DMA,
            pltpu.SemaphoreType.DMA,
        ],
    )
    return f(x, batch).reshape(_G, _D)


# speedup vs baseline: 1.1956x; 1.0172x over previous
import jax
import jax.numpy as jnp
from jax import lax
from jax.experimental import pallas as pl
from jax.experimental.pallas import tpu as pltpu
from jax.experimental.pallas import tpu_sc as plsc

_N = 50000
_D = 256
_G = 64
_L = 16
_NV = _D // _L
_C = 128
_NEG = float("-inf")


def _lane_extract(vec, lane):
    lid = lax.broadcasted_iota(jnp.int32, (_L,), 0)
    return jnp.max(jnp.where(lid == lane, vec, 0))


def _sc_body(x_hbm, batch_hbm, out_hbm, batch_v, buf0, buf1, accv, sem0, sem1):
    wid = lax.axis_index("s") * 2 + lax.axis_index("c")

    pltpu.sync_copy(batch_hbm, batch_v)
    lid = lax.broadcasted_iota(jnp.int32, (_L,), 0)
    tgt = 2 * wid + lid
    lo = jnp.zeros((_L,), jnp.int32)
    hi = jnp.full((_L,), _N, jnp.int32)

    def bs_step(_, carry):
        lo, hi = carry
        active = lo < hi
        mid = lax.shift_right_logical(lo + hi, 1)
        v = plsc.load_gather(batch_v, [jnp.minimum(mid, _N - 1)])
        less = v < tgt
        lo = jnp.where(active & less, mid + 1, lo)
        hi = jnp.where(active & (~less), mid, hi)
        return lo, hi

    lo, _ = lax.fori_loop(0, 16, bs_step, (lo, hi))
    s0 = _lane_extract(lo, 0)
    sm = _lane_extract(lo, 1)
    s1 = _lane_extract(lo, 2)

    a0 = jnp.bitwise_and(s0, -8)
    top = jnp.bitwise_and(s1 + 7, -8)
    nchunks = (top - a0 + _C - 1) // _C

    def off(k):
        o = jnp.maximum(0, jnp.minimum(a0 + k * _C, top - _C))
        return pl.multiple_of(o, 8)

    def start(k, buf, sem):
        pltpu.make_async_copy(x_hbm.at[pl.ds(off(k), _C)], buf, sem).start()

    @pl.when(nchunks > 0)
    def _p0():
        start(0, buf0, sem0)

    @pl.when(nchunks > 1)
    def _p1():
        start(1, buf1, sem1)

    neg = jnp.full((_L,), _NEG, jnp.float32)
    acc = [neg] * (2 * _NV)

    def make_row_body(buf, base):
        def row_body(r, a):
            a = list(a)
            for j in range(_NV):
                a[base + j] = jnp.maximum(a[base + j], buf[r, pl.ds(j * _L, _L)])
            return tuple(a)
        return row_body

    def chunk(k, buf, sem, acc):
        @pl.when(k < nchunks)
        def _w():
            pltpu.make_async_copy(
                x_hbm.at[pl.ds(off(k), _C)], buf, sem).wait()

        valid = k < nchunks
        o = off(k)
        i0 = jnp.where(valid, jnp.clip(s0 - o, 0, _C), 0)
        i1 = jnp.where(valid, jnp.clip(sm - o, 0, _C), 0)
        i2 = jnp.where(valid, jnp.clip(s1 - o, 0, _C), 0)
        acc = lax.fori_loop(i0, i1, make_row_body(buf, 0), acc)
        acc = lax.fori_loop(i1, i2, make_row_body(buf, _NV), acc)

        @pl.when(k + 2 < nchunks)
        def _n():
            start(k + 2, buf, sem)

        return acc

    def pair(p, acc):
        acc = chunk(2 * p, buf0, sem0, acc)
        acc = chunk(2 * p + 1, buf1, sem1, acc)
        return acc

    acc = lax.fori_loop(0, (nchunks + 1) // 2, pair, tuple(acc))

    for j in range(_NV):
        accv[pl.ds(j * _L, _L)] = acc[j]
        accv[pl.ds(_D + j * _L, _L)] = acc[_NV + j]
    pltpu.sync_copy(accv, out_hbm.at[pl.ds(wid * 2 * _D, 2 * _D)])


@jax.jit
def kernel(x, batch):
    mesh = plsc.VectorSubcoreMesh(core_axis_name="c", subcore_axis_name="s")
    f = pl.kernel(
        _sc_body,
        out_type=jax.ShapeDtypeStruct((_G * _D,), jnp.float32),
        mesh=mesh,
        compiler_params=pltpu.CompilerParams(needs_layout_passes=False),
        scratch_types=[
            pltpu.VMEM((_N,), jnp.int32),
            pltpu.VMEM((_C, _D), jnp.float32),
            pltpu.VMEM((_C, _D), jnp.float32),
            pltpu.VMEM((2 * _D,), jnp.float32),
            pltpu.SemaphoreType.DMA,
            pltpu.SemaphoreType.DMA,
        ],
    )
    return f(x, batch).reshape(_G, _D)


# speedup vs baseline: 1.2777x; 1.0686x over previous
import jax
import jax.numpy as jnp
from jax import lax
from jax.experimental import pallas as pl
from jax.experimental.pallas import tpu as pltpu
from jax.experimental.pallas import tpu_sc as plsc

_N = 50000
_D = 256
_G = 64
_L = 16
_NV = _D // _L
_C = 128
_NEG = float("-inf")


def _lane_extract(vec, lane):
    lid = lax.broadcasted_iota(jnp.int32, (_L,), 0)
    return jnp.max(jnp.where(lid == lane, vec, 0))


def _sc_body(x_hbm, batch_hbm, out_hbm, batch_v, bsh, buf0, buf1, accv,
             sem0, sem1):
    sid = lax.axis_index("s")
    wid = sid * 2 + lax.axis_index("c")

    @pl.when(sid == 0)
    def _stage():
        pltpu.sync_copy(batch_hbm, bsh)

    plsc.subcore_barrier()
    pltpu.sync_copy(bsh, batch_v)
    lid = lax.broadcasted_iota(jnp.int32, (_L,), 0)
    tgt = 2 * wid + lid
    lo = jnp.zeros((_L,), jnp.int32)
    hi = jnp.full((_L,), _N, jnp.int32)

    def bs_step(_, carry):
        lo, hi = carry
        active = lo < hi
        mid = lax.shift_right_logical(lo + hi, 1)
        v = plsc.load_gather(batch_v, [jnp.minimum(mid, _N - 1)])
        less = v < tgt
        lo = jnp.where(active & less, mid + 1, lo)
        hi = jnp.where(active & (~less), mid, hi)
        return lo, hi

    lo, _ = lax.fori_loop(0, 16, bs_step, (lo, hi))
    s0 = _lane_extract(lo, 0)
    sm = _lane_extract(lo, 1)
    s1 = _lane_extract(lo, 2)

    a0 = jnp.bitwise_and(s0, -8)
    top = jnp.bitwise_and(s1 + 7, -8)
    nchunks = (top - a0 + _C - 1) // _C

    def off(k):
        o = jnp.maximum(0, jnp.minimum(a0 + k * _C, top - _C))
        return pl.multiple_of(o, 8)

    def start(k, buf, sem):
        pltpu.make_async_copy(x_hbm.at[pl.ds(off(k), _C)], buf, sem).start()

    @pl.when(nchunks > 0)
    def _p0():
        start(0, buf0, sem0)

    @pl.when(nchunks > 1)
    def _p1():
        start(1, buf1, sem1)

    neg = jnp.full((_L,), _NEG, jnp.float32)
    acc = [neg] * (2 * _NV)

    def make_row_body(buf, base):
        def row_body(r, a):
            a = list(a)
            for j in range(_NV):
                a[base + j] = jnp.maximum(a[base + j], buf[r, pl.ds(j * _L, _L)])
            return tuple(a)
        return row_body

    def chunk(k, buf, sem, acc):
        @pl.when(k < nchunks)
        def _w():
            pltpu.make_async_copy(
                x_hbm.at[pl.ds(off(k), _C)], buf, sem).wait()

        valid = k < nchunks
        o = off(k)
        i0 = jnp.where(valid, jnp.clip(s0 - o, 0, _C), 0)
        i1 = jnp.where(valid, jnp.clip(sm - o, 0, _C), 0)
        i2 = jnp.where(valid, jnp.clip(s1 - o, 0, _C), 0)
        acc = lax.fori_loop(i0, i1, make_row_body(buf, 0), acc)
        acc = lax.fori_loop(i1, i2, make_row_body(buf, _NV), acc)

        @pl.when(k + 2 < nchunks)
        def _n():
            start(k + 2, buf, sem)

        return acc

    def pair(p, acc):
        acc = chunk(2 * p, buf0, sem0, acc)
        acc = chunk(2 * p + 1, buf1, sem1, acc)
        return acc

    acc = lax.fori_loop(0, (nchunks + 1) // 2, pair, tuple(acc))

    for j in range(_NV):
        accv[pl.ds(j * _L, _L)] = acc[j]
        accv[pl.ds(_D + j * _L, _L)] = acc[_NV + j]
    pltpu.sync_copy(accv, out_hbm.at[pl.ds(wid * 2 * _D, 2 * _D)])


@jax.jit
def kernel(x, batch):
    mesh = plsc.VectorSubcoreMesh(core_axis_name="c", subcore_axis_name="s")
    f = pl.kernel(
        _sc_body,
        out_type=jax.ShapeDtypeStruct((_G * _D,), jnp.float32),
        mesh=mesh,
        compiler_params=pltpu.CompilerParams(needs_layout_passes=False),
        scratch_types=[
            pltpu.VMEM((_N,), jnp.int32),
            pltpu.VMEM_SHARED((_N,), jnp.int32),
            pltpu.VMEM((_C, _D), jnp.float32),
            pltpu.VMEM((_C, _D), jnp.float32),
            pltpu.VMEM((2 * _D,), jnp.float32),
            pltpu.SemaphoreType.DMA,
            pltpu.SemaphoreType.DMA,
        ],
    )
    return f(x, batch).reshape(_G, _D)
